# baseline (speedup 1.0000x reference)
import jax
import jax.numpy as jnp
from jax.experimental import pallas as pl


def _ln(z, g, bt):
    mu = z.mean(-1, keepdims=True)
    var = z.var(-1, keepdims=True)
    return (z - mu) / jnp.sqrt(var + 1e-5) * g + bt


def _bn(z, g, bt):
    mu = z.mean(0)
    var = z.var(0)
    return (z - mu) / jnp.sqrt(var + 1e-5) * g + bt


def _copy_body(x_ref, o_ref):
    o_ref[...] = x_ref[...]


def kernel(x, edge_index, edge_attr, batch, params):
    src = edge_index[0]
    dst = edge_index[1]
    h = x
    outs = [x]
    for p in params["convs"]:
        e = edge_attr @ p["We"] + p["be"]
        m = jax.nn.relu(h[src] + e)
        agg = jax.ops.segment_sum(m, dst, num_segments=10000)
        z = (h + agg) @ p["W"] + p["b"]
        z = jax.nn.relu(z)
        h = _ln(z, p["g"], p["bt"])
        outs.append(h)
    xc = jnp.concatenate(outs, axis=-1)
    h1 = _bn(jax.nn.relu(xc @ params["fc1"]["W"] + params["fc1"]["b"]),
             params["fc1"]["g"], params["fc1"]["bt"])
    h2 = _bn(jax.nn.relu(h1 @ params["fc2"]["W"] + params["fc2"]["b"]),
             params["fc2"]["g"], params["fc2"]["bt"])
    h2 = pl.pallas_call(
        _copy_body,
        out_shape=jax.ShapeDtypeStruct(h2.shape, h2.dtype),
    )(h2)
    edge_probs = jax.nn.sigmoid(h2[src] @ params["pol"]["W"] + params["pol"]["b"])[:, 0]
    sums = jax.ops.segment_sum(h2, batch, num_segments=64)
    counts = jax.ops.segment_sum(jnp.ones((10000, 1), jnp.float32), batch, num_segments=64)
    pooled = sums / jnp.maximum(counts, 1.0)
    value = jnp.tanh(pooled @ params["val"]["W"] + params["val"]["b"])[:, 0]
    return edge_probs, value


# speedup vs baseline: 1.7318x; 1.7318x over previous
import functools

import jax
import jax.numpy as jnp
from jax import lax
from jax.experimental import pallas as pl
from jax.experimental.pallas import tpu as pltpu
from jax.experimental.pallas import tpu_sc as plsc

N = 10000
E = 320000
F_IN = 128
C = 256
G = 64

NC = 2
NS = 16
NT = NC * NS
NPT = (N + NT - 1) // NT
NPAD = NT * NPT

CH = 3200
PW = E + CH + 16
CE = 64

_MESH = plsc.VectorSubcoreMesh(core_axis_name="c", subcore_axis_name="s")


def _wid():
    return lax.axis_index("s") * NC + lax.axis_index("c")


@functools.partial(
    pl.kernel,
    out_type=(
        jax.ShapeDtypeStruct((NT * PW,), jnp.int32),
        jax.ShapeDtypeStruct((NT * PW,), jnp.int32),
        jax.ShapeDtypeStruct((NT * PW,), jnp.float32),
        jax.ShapeDtypeStruct((NT * 16,), jnp.int32),
    ),
    mesh=_MESH,
    scratch_types=[
        pltpu.VMEM((CH,), jnp.int32),
        pltpu.VMEM((CH,), jnp.int32),
        pltpu.VMEM((CH,), jnp.float32),
        pltpu.VMEM((CH + 16,), jnp.int32),
        pltpu.VMEM((CH + 16,), jnp.int32),
        pltpu.VMEM((CH + 16,), jnp.float32),
        pltpu.VMEM((16,), jnp.int32),
    ],
    compiler_params=pltpu.CompilerParams(needs_layout_passes=False),
)
def _partition(dst_h, src_h, attr_h, osrc, odl, oattr, ocnt,
               dbuf, sbuf, abuf, st_s, st_d, st_a, cbuf):
    t = _wid()
    lo = t * NPT
    hi = lo + NPT
    zi = jnp.zeros((16,), jnp.int32)
    zf = jnp.zeros((16,), jnp.float32)

    def memset(i, _):
        st_s[pl.ds(i * 16, 16)] = zi
        st_d[pl.ds(i * 16, 16)] = zi
        st_a[pl.ds(i * 16, 16)] = zf
        return 0

    lax.fori_loop(0, (CH + 16) // 16, memset, 0)

    def chunk_body(c, carry):
        off, total = carry
        pltpu.sync_copy(dst_h.at[pl.ds(c * CH, CH)], dbuf)
        pltpu.sync_copy(src_h.at[pl.ds(c * CH, CH)], sbuf)
        pltpu.sync_copy(attr_h.at[pl.ds(c * CH, CH)], abuf)

        def vec_body(v, off):
            d = dbuf[pl.ds(v * 16, 16)]
            s = sbuf[pl.ds(v * 16, 16)]
            a = abuf[pl.ds(v * 16, 16)]
            m = (d >= lo) & (d < hi)
            pc = plsc.cumsum(jnp.where(m, 1, 0).astype(jnp.int32))
            pos = pc + (off - 1)
            plsc.store_scatter(st_s, [pos], s, mask=m)
            plsc.store_scatter(st_d, [pos], d - lo, mask=m)
            plsc.store_scatter(st_a, [pos], a, mask=m)
            return off + pc[15]

        off = lax.fori_loop(0, CH // 16, vec_body, off)
        fl = jnp.bitwise_and(off, -16)
        to = pl.multiple_of(t * PW + total, 16)
        pltpu.sync_copy(st_s.at[pl.ds(0, CH)], osrc.at[pl.ds(to, CH)])
        pltpu.sync_copy(st_d.at[pl.ds(0, CH)], odl.at[pl.ds(to, CH)])
        pltpu.sync_copy(st_a.at[pl.ds(0, CH)], oattr.at[pl.ds(to, CH)])
        rs = st_s[pl.ds(fl, 16)]
        rd = st_d[pl.ds(fl, 16)]
        ra = st_a[pl.ds(fl, 16)]
        st_s[pl.ds(0, 16)] = rs
        st_d[pl.ds(0, 16)] = rd
        st_a[pl.ds(0, 16)] = ra
        return off - fl, total + fl

    off, total = lax.fori_loop(0, E // CH, chunk_body, (jnp.int32(0), jnp.int32(0)))
    to = pl.multiple_of(t * PW + total, 16)
    pltpu.sync_copy(st_s.at[pl.ds(0, 16)], osrc.at[pl.ds(to, 16)])
    pltpu.sync_copy(st_d.at[pl.ds(0, 16)], odl.at[pl.ds(to, 16)])
    pltpu.sync_copy(st_a.at[pl.ds(0, 16)], oattr.at[pl.ds(to, 16)])
    cbuf[...] = jnp.broadcast_to(total + off, (16,))
    pltpu.sync_copy(cbuf, ocnt.at[pl.ds(pl.multiple_of(t * 16, 16), 16)])


def _make_edge_kernel(F):
    J = F // 16

    @functools.partial(
        pl.kernel,
        out_type=jax.ShapeDtypeStruct((NPAD * F,), jnp.float32),
        mesh=_MESH,
        scratch_types=[
            pltpu.VMEM((NPT * F,), jnp.float32),
            pltpu.VMEM((2, CE), jnp.int32),
            pltpu.VMEM((2, CE + 16), jnp.int32),
            pltpu.VMEM((2, CE + 16), jnp.float32),
            pltpu.VMEM((2, CE, F), jnp.float32),
            pltpu.VMEM((16,), jnp.int32),
            pltpu.VMEM((F,), jnp.float32),
            pltpu.VMEM((F,), jnp.float32),
            pltpu.SemaphoreType.DMA,
            pltpu.SemaphoreType.DMA,
        ],
    )
    def edge_kernel(table_h, psrc, pdl, pattr, cnt_h, we_h, be_h, out_h,
                    acc, sidx, dlb, atb, rows, cbuf, web, beb, sem0, sem1):
        t = _wid()
        pltpu.sync_copy(cnt_h.at[pl.ds(pl.multiple_of(t * 16, 16), 16)], cbuf)
        pltpu.sync_copy(we_h, web)
        pltpu.sync_copy(be_h, beb)
        k = cbuf[...][0]
        nch = (k + (CE - 1)) // CE

        zf = jnp.zeros((16,), jnp.float32)

        def zero_body(r, _):
            rb = pl.multiple_of(r * F, 16)
            for j in range(J):
                acc[pl.ds(rb + 16 * j, 16)] = zf
            return 0

        lax.fori_loop(0, NPT, zero_body, 0)

        sems = (sem0, sem1)

        def start(c, b):
            cb = pl.multiple_of(t * PW + c * CE, 16)
            pltpu.sync_copy(psrc.at[pl.ds(cb, CE)], sidx.at[b])
            pltpu.sync_copy(pdl.at[pl.ds(cb, CE)], dlb.at[b, pl.ds(0, CE)])
            pltpu.sync_copy(pattr.at[pl.ds(cb, CE)], atb.at[b, pl.ds(0, CE)])
            pltpu.make_async_copy(table_h.at[sidx.at[b]], rows.at[b], sems[b]).start()

        @pl.when(nch > 0)
        def _():
            start(0, 0)

        @pl.when(nch > 1)
        def _():
            start(1, 1)

        wes = [web[pl.ds(16 * j, 16)] for j in range(J)]
        bes = [beb[pl.ds(16 * j, 16)] for j in range(J)]

        def pair_body(c2, _):
            for b in range(2):
                c = 2 * c2 + b

                @pl.when(c < nch)
                def _():
                    pltpu.make_async_copy(
                        table_h.at[sidx.at[b]], rows.at[b], sems[b]).wait()
                    ce_k = jnp.minimum(CE, k - c * CE)

                    def edge_body(e, _):
                        dl = dlb[b, pl.ds(e, 16)][0]
                        a = atb[b, pl.ds(e, 16)][0]
                        db = pl.multiple_of(dl * F, 16)
                        for j in range(J):
                            g = rows[b, e, pl.ds(16 * j, 16)]
                            m = jnp.maximum(g + (a * wes[j] + bes[j]), 0.0)
                            plsc.addupdate(acc.at[pl.ds(db + 16 * j, 16)], m)
                        return 0

                    lax.fori_loop(0, ce_k, edge_body, 0)

                    @pl.when(c + 2 < nch)
                    def _():
                        start(c + 2, b)
            return 0

        lax.fori_loop(0, (nch + 1) // 2, pair_body, 0)
        pltpu.sync_copy(acc, out_h.at[pl.ds(pl.multiple_of(t * NPT * F, 16), NPT * F)])

    return edge_kernel


_edge_128 = _make_edge_kernel(128)
_edge_256 = _make_edge_kernel(256)


def _ln(z, g, bt):
    mu = z.mean(-1, keepdims=True)
    var = z.var(-1, keepdims=True)
    return (z - mu) / jnp.sqrt(var + 1e-5) * g + bt


def _bn(z, g, bt):
    mu = z.mean(0)
    var = z.var(0)
    return (z - mu) / jnp.sqrt(var + 1e-5) * g + bt


def kernel(x, edge_index, edge_attr, batch, params):
    src = edge_index[0]
    dst = edge_index[1]
    attr = edge_attr[:, 0]

    psrc, pdl, pattr, pcnt = _partition(dst, src, attr)

    h = x
    outs = [x]
    for i, p in enumerate(params["convs"]):
        F = h.shape[1]
        ek = _edge_128 if F == 128 else _edge_256
        agg = ek(h, psrc, pdl, pattr, pcnt, p["We"][0], p["be"])
        agg = agg.reshape(NPAD, F)[:N]
        z = (h + agg) @ p["W"] + p["b"]
        z = jax.nn.relu(z)
        h = _ln(z, p["g"], p["bt"])
        outs.append(h)

    xc = jnp.concatenate(outs, axis=-1)
    h1 = _bn(jax.nn.relu(xc @ params["fc1"]["W"] + params["fc1"]["b"]),
             params["fc1"]["g"], params["fc1"]["bt"])
    h2 = _bn(jax.nn.relu(h1 @ params["fc2"]["W"] + params["fc2"]["b"]),
             params["fc2"]["g"], params["fc2"]["bt"])
    edge_probs = jax.nn.sigmoid(h2[src] @ params["pol"]["W"] + params["pol"]["b"])[:, 0]
    sums = jax.ops.segment_sum(h2, batch, num_segments=G)
    counts = jax.ops.segment_sum(jnp.ones((N, 1), jnp.float32), batch, num_segments=G)
    pooled = sums / jnp.maximum(counts, 1.0)
    value = jnp.tanh(pooled @ params["val"]["W"] + params["val"]["b"])[:, 0]
    return edge_probs, value


# speedup vs baseline: 2.0139x; 1.1629x over previous
import functools

import jax
import jax.numpy as jnp
from jax import lax
from jax.experimental import pallas as pl
from jax.experimental.pallas import tpu as pltpu
from jax.experimental.pallas import tpu_sc as plsc

N = 10000
E = 320000
F_IN = 128
C = 256
G = 64

NC = 2
NS = 16
NT = NC * NS
NPT = (N + NT - 1) // NT
NPAD = NT * NPT

CH = 3200
PW = E + CH + 16
CE = 64

_MESH = plsc.VectorSubcoreMesh(core_axis_name="c", subcore_axis_name="s")


def _wid():
    return lax.axis_index("s") * NC + lax.axis_index("c")


@functools.partial(
    pl.kernel,
    out_type=(
        jax.ShapeDtypeStruct((NT * PW,), jnp.int32),
        jax.ShapeDtypeStruct((NT * PW,), jnp.int32),
        jax.ShapeDtypeStruct((NT * PW,), jnp.float32),
        jax.ShapeDtypeStruct((NT * 16,), jnp.int32),
    ),
    mesh=_MESH,
    scratch_types=[
        pltpu.VMEM((CH,), jnp.int32),
        pltpu.VMEM((CH,), jnp.int32),
        pltpu.VMEM((CH,), jnp.float32),
        pltpu.VMEM((CH + 16,), jnp.int32),
        pltpu.VMEM((CH + 16,), jnp.int32),
        pltpu.VMEM((CH + 16,), jnp.float32),
        pltpu.VMEM((16,), jnp.int32),
    ],
    compiler_params=pltpu.CompilerParams(needs_layout_passes=False),
)
def _partition(dst_h, src_h, attr_h, osrc, odl, oattr, ocnt,
               dbuf, sbuf, abuf, st_s, st_d, st_a, cbuf):
    t = _wid()
    lo = t * NPT
    hi = lo + NPT
    zi = jnp.zeros((16,), jnp.int32)
    zf = jnp.zeros((16,), jnp.float32)

    def memset(i, _):
        st_s[pl.ds(i * 16, 16)] = zi
        st_d[pl.ds(i * 16, 16)] = zi
        st_a[pl.ds(i * 16, 16)] = zf
        return 0

    lax.fori_loop(0, (CH + 16) // 16, memset, 0)

    def chunk_body(c, carry):
        off, total = carry
        pltpu.sync_copy(dst_h.at[pl.ds(c * CH, CH)], dbuf)
        pltpu.sync_copy(src_h.at[pl.ds(c * CH, CH)], sbuf)
        pltpu.sync_copy(attr_h.at[pl.ds(c * CH, CH)], abuf)

        def vec_body(v, off):
            d = dbuf[pl.ds(v * 16, 16)]
            s = sbuf[pl.ds(v * 16, 16)]
            a = abuf[pl.ds(v * 16, 16)]
            m = (d >= lo) & (d < hi)
            pc = plsc.cumsum(jnp.where(m, 1, 0).astype(jnp.int32))
            pos = pc + (off - 1)
            plsc.store_scatter(st_s, [pos], s, mask=m)
            plsc.store_scatter(st_d, [pos], d - lo, mask=m)
            plsc.store_scatter(st_a, [pos], a, mask=m)
            return off + pc[15]

        off = lax.fori_loop(0, CH // 16, vec_body, off)
        fl = jnp.bitwise_and(off, -16)
        to = pl.multiple_of(t * PW + total, 16)
        pltpu.sync_copy(st_s.at[pl.ds(0, CH)], osrc.at[pl.ds(to, CH)])
        pltpu.sync_copy(st_d.at[pl.ds(0, CH)], odl.at[pl.ds(to, CH)])
        pltpu.sync_copy(st_a.at[pl.ds(0, CH)], oattr.at[pl.ds(to, CH)])
        rs = st_s[pl.ds(fl, 16)]
        rd = st_d[pl.ds(fl, 16)]
        ra = st_a[pl.ds(fl, 16)]
        st_s[pl.ds(0, 16)] = rs
        st_d[pl.ds(0, 16)] = rd
        st_a[pl.ds(0, 16)] = ra
        return off - fl, total + fl

    off, total = lax.fori_loop(0, E // CH, chunk_body, (jnp.int32(0), jnp.int32(0)))
    to = pl.multiple_of(t * PW + total, 16)
    pltpu.sync_copy(st_s.at[pl.ds(0, 16)], osrc.at[pl.ds(to, 16)])
    pltpu.sync_copy(st_d.at[pl.ds(0, 16)], odl.at[pl.ds(to, 16)])
    pltpu.sync_copy(st_a.at[pl.ds(0, 16)], oattr.at[pl.ds(to, 16)])
    cbuf[...] = jnp.broadcast_to(total + off, (16,))
    pltpu.sync_copy(cbuf, ocnt.at[pl.ds(pl.multiple_of(t * 16, 16), 16)])


SB = 2048
SPC = SB // CE


def _make_edge_kernel(F):
    J = F // 16

    @functools.partial(
        pl.kernel,
        out_type=jax.ShapeDtypeStruct((NPAD * F,), jnp.float32),
        mesh=_MESH,
        scratch_types=[
            pltpu.VMEM((NPT * F,), jnp.float32),
            pltpu.VMEM((SB,), jnp.int32),
            pltpu.VMEM((SB + 16,), jnp.int32),
            pltpu.VMEM((SB + 16,), jnp.float32),
            pltpu.VMEM((2, CE, F), jnp.float32),
            pltpu.VMEM((16,), jnp.int32),
            pltpu.VMEM((F,), jnp.float32),
            pltpu.VMEM((F,), jnp.float32),
            pltpu.SemaphoreType.DMA,
            pltpu.SemaphoreType.DMA,
        ],
    )
    def edge_kernel(table_h, psrc, pdl, pattr, cnt_h, we_h, be_h, out_h,
                    acc, sidx, dlb, atb, rows, cbuf, web, beb, sem0, sem1):
        t = _wid()
        pltpu.sync_copy(cnt_h.at[pl.ds(pl.multiple_of(t * 16, 16), 16)], cbuf)
        pltpu.sync_copy(we_h, web)
        pltpu.sync_copy(be_h, beb)
        k = cbuf[...][0]

        zf = jnp.zeros((16,), jnp.float32)
        zi = jnp.zeros((16,), jnp.int32)

        def idx_clear(i, _):
            sidx[pl.ds(i * 16, 16)] = zi
            return 0

        lax.fori_loop(0, SB // 16, idx_clear, 0)

        def zero_body(r, _):
            rb = pl.multiple_of(r * F, 16)
            for j in range(J):
                acc[pl.ds(rb + 16 * j, 16)] = zf
            return 0

        lax.fori_loop(0, NPT, zero_body, 0)

        sems = (sem0, sem1)

        def start(c, b):
            pltpu.make_async_copy(
                table_h.at[sidx.at[pl.ds(pl.multiple_of(c * CE, 8), CE)]],
                rows.at[b], sems[b]).start()

        wes = [web[pl.ds(16 * j, 16)] for j in range(J)]
        bes = [beb[pl.ds(16 * j, 16)] for j in range(J)]

        def super_body(si, _):
            sb0 = si * SB
            hb = pl.multiple_of(t * PW + sb0, 16)
            pltpu.sync_copy(psrc.at[pl.ds(hb, SB)], sidx)
            pltpu.sync_copy(pdl.at[pl.ds(hb, SB)], dlb.at[pl.ds(0, SB)])
            pltpu.sync_copy(pattr.at[pl.ds(hb, SB)], atb.at[pl.ds(0, SB)])
            sck = jnp.minimum(SB, k - sb0)
            nch = (sck + (CE - 1)) // CE

            @pl.when(nch > 0)
            def _():
                start(0, 0)

            @pl.when(nch > 1)
            def _():
                start(1, 1)

            def pair_body(c2, _):
                for b in range(2):
                    c = 2 * c2 + b

                    @pl.when(c < nch)
                    def _():
                        pltpu.make_async_copy(
                            table_h.at[sidx.at[pl.ds(pl.multiple_of(c * CE, 8), CE)]],
                            rows.at[b], sems[b]).wait()
                        ce_k = jnp.minimum(CE, sck - c * CE)
                        e0 = c * CE

                        def edge_body(e, _):
                            dl = dlb[pl.ds(e0 + e, 16)][0]
                            a = atb[pl.ds(e0 + e, 16)][0]
                            db = pl.multiple_of(dl * F, 16)
                            for j in range(J):
                                g = rows[b, e, pl.ds(16 * j, 16)]
                                m = jnp.maximum(g + (a * wes[j] + bes[j]), 0.0)
                                plsc.addupdate(acc.at[pl.ds(db + 16 * j, 16)], m)
                            return 0

                        lax.fori_loop(0, ce_k, edge_body, 0)

                        @pl.when(c + 2 < nch)
                        def _():
                            start(c + 2, b)
                return 0

            lax.fori_loop(0, (nch + 1) // 2, pair_body, 0)
            return 0

        lax.fori_loop(0, (k + (SB - 1)) // SB, super_body, 0)
        pltpu.sync_copy(acc, out_h.at[pl.ds(pl.multiple_of(t * NPT * F, 16), NPT * F)])

    return edge_kernel


_edge_128 = _make_edge_kernel(128)
_edge_256 = _make_edge_kernel(256)


def _ln(z, g, bt):
    mu = z.mean(-1, keepdims=True)
    var = z.var(-1, keepdims=True)
    return (z - mu) / jnp.sqrt(var + 1e-5) * g + bt


def _bn(z, g, bt):
    mu = z.mean(0)
    var = z.var(0)
    return (z - mu) / jnp.sqrt(var + 1e-5) * g + bt


def kernel(x, edge_index, edge_attr, batch, params):
    src = edge_index[0]
    dst = edge_index[1]
    attr = edge_attr[:, 0]

    psrc, pdl, pattr, pcnt = _partition(dst, src, attr)

    h = x
    outs = [x]
    for i, p in enumerate(params["convs"]):
        F = h.shape[1]
        ek = _edge_128 if F == 128 else _edge_256
        agg = ek(h, psrc, pdl, pattr, pcnt, p["We"][0], p["be"])
        agg = agg.reshape(NPAD, F)[:N]
        z = (h + agg) @ p["W"] + p["b"]
        z = jax.nn.relu(z)
        h = _ln(z, p["g"], p["bt"])
        outs.append(h)

    xc = jnp.concatenate(outs, axis=-1)
    h1 = _bn(jax.nn.relu(xc @ params["fc1"]["W"] + params["fc1"]["b"]),
             params["fc1"]["g"], params["fc1"]["bt"])
    h2 = _bn(jax.nn.relu(h1 @ params["fc2"]["W"] + params["fc2"]["b"]),
             params["fc2"]["g"], params["fc2"]["bt"])
    edge_probs = jax.nn.sigmoid(h2[src] @ params["pol"]["W"] + params["pol"]["b"])[:, 0]
    sums = jax.ops.segment_sum(h2, batch, num_segments=G)
    counts = jax.ops.segment_sum(jnp.ones((N, 1), jnp.float32), batch, num_segments=G)
    pooled = sums / jnp.maximum(counts, 1.0)
    value = jnp.tanh(pooled @ params["val"]["W"] + params["val"]["b"])[:, 0]
    return edge_probs, value


# speedup vs baseline: 4.5775x; 2.2730x over previous
import functools

import jax
import jax.numpy as jnp
from jax import lax
from jax.experimental import pallas as pl
from jax.experimental.pallas import tpu as pltpu
from jax.experimental.pallas import tpu_sc as plsc

N = 10000
E = 320000
F_IN = 128
C = 256
G = 64

NC = 2
NS = 16
NT = NC * NS
NPT = (N + NT - 1) // NT
NPAD = NT * NPT

CH = 3200
PW = E + CH + 16
CE = 64

_MESH = plsc.VectorSubcoreMesh(core_axis_name="c", subcore_axis_name="s")


def _wid():
    return lax.axis_index("s") * NC + lax.axis_index("c")


@functools.partial(
    pl.kernel,
    out_type=(
        jax.ShapeDtypeStruct((NT * PW,), jnp.int32),
        jax.ShapeDtypeStruct((NT * PW,), jnp.int32),
        jax.ShapeDtypeStruct((NT * PW,), jnp.float32),
        jax.ShapeDtypeStruct((NT * 16,), jnp.int32),
    ),
    mesh=_MESH,
    scratch_types=[
        pltpu.VMEM((CH,), jnp.int32),
        pltpu.VMEM((CH,), jnp.int32),
        pltpu.VMEM((CH,), jnp.float32),
        pltpu.VMEM((CH + 16,), jnp.int32),
        pltpu.VMEM((CH + 16,), jnp.int32),
        pltpu.VMEM((CH + 16,), jnp.float32),
        pltpu.VMEM((16,), jnp.int32),
    ],
    compiler_params=pltpu.CompilerParams(needs_layout_passes=False),
)
def _partition(dst_h, src_h, attr_h, osrc, odl, oattr, ocnt,
               dbuf, sbuf, abuf, st_s, st_d, st_a, cbuf):
    t = _wid()
    lo = t * NPT
    hi = lo + NPT
    zi = jnp.zeros((16,), jnp.int32)
    zf = jnp.zeros((16,), jnp.float32)

    def memset(i, _):
        st_s[pl.ds(i * 16, 16)] = zi
        st_d[pl.ds(i * 16, 16)] = zi
        st_a[pl.ds(i * 16, 16)] = zf
        return 0

    lax.fori_loop(0, (CH + 16) // 16, memset, 0)

    def chunk_body(c, carry):
        off, total = carry
        pltpu.sync_copy(dst_h.at[pl.ds(c * CH, CH)], dbuf)
        pltpu.sync_copy(src_h.at[pl.ds(c * CH, CH)], sbuf)
        pltpu.sync_copy(attr_h.at[pl.ds(c * CH, CH)], abuf)

        def vec_body(v, off):
            d = dbuf[pl.ds(v * 16, 16)]
            s = sbuf[pl.ds(v * 16, 16)]
            a = abuf[pl.ds(v * 16, 16)]
            m = (d >= lo) & (d < hi)
            pc = plsc.cumsum(jnp.where(m, 1, 0).astype(jnp.int32))
            pos = pc + (off - 1)
            plsc.store_scatter(st_s, [pos], s, mask=m)
            plsc.store_scatter(st_d, [pos], d - lo, mask=m)
            plsc.store_scatter(st_a, [pos], a, mask=m)
            return off + pc[15]

        off = lax.fori_loop(0, CH // 16, vec_body, off)
        fl = jnp.bitwise_and(off, -16)
        to = pl.multiple_of(t * PW + total, 16)
        pltpu.sync_copy(st_s.at[pl.ds(0, CH)], osrc.at[pl.ds(to, CH)])
        pltpu.sync_copy(st_d.at[pl.ds(0, CH)], odl.at[pl.ds(to, CH)])
        pltpu.sync_copy(st_a.at[pl.ds(0, CH)], oattr.at[pl.ds(to, CH)])
        rs = st_s[pl.ds(fl, 16)]
        rd = st_d[pl.ds(fl, 16)]
        ra = st_a[pl.ds(fl, 16)]
        st_s[pl.ds(0, 16)] = rs
        st_d[pl.ds(0, 16)] = rd
        st_a[pl.ds(0, 16)] = ra
        return off - fl, total + fl

    off, total = lax.fori_loop(0, E // CH, chunk_body, (jnp.int32(0), jnp.int32(0)))
    to = pl.multiple_of(t * PW + total, 16)
    pltpu.sync_copy(st_s.at[pl.ds(0, 16)], osrc.at[pl.ds(to, 16)])
    pltpu.sync_copy(st_d.at[pl.ds(0, 16)], odl.at[pl.ds(to, 16)])
    pltpu.sync_copy(st_a.at[pl.ds(0, 16)], oattr.at[pl.ds(to, 16)])
    cbuf[...] = jnp.broadcast_to(total + off, (16,))
    pltpu.sync_copy(cbuf, ocnt.at[pl.ds(pl.multiple_of(t * 16, 16), 16)])


SB = 2048
SPC = SB // CE


def _make_edge_kernel(F):
    J = F // 16

    @functools.partial(
        pl.kernel,
        out_type=jax.ShapeDtypeStruct((NPAD * F,), jnp.float32),
        mesh=_MESH,
        scratch_types=[
            pltpu.VMEM((NPT * F,), jnp.float32),
            pltpu.VMEM((SB,), jnp.int32),
            pltpu.VMEM((SB + 16,), jnp.int32),
            pltpu.VMEM((SB + 16,), jnp.float32),
            pltpu.VMEM((2, CE, F), jnp.float32),
            pltpu.VMEM((16,), jnp.int32),
            pltpu.VMEM((F,), jnp.float32),
            pltpu.SemaphoreType.DMA,
            pltpu.SemaphoreType.DMA,
        ],
    )
    def edge_kernel(table_h, psrc, pdl, pattr, cnt_h, we_h, out_h,
                    acc, sidx, dlb, atb, rows, cbuf, web, sem0, sem1):
        t = _wid()
        pltpu.sync_copy(cnt_h.at[pl.ds(pl.multiple_of(t * 16, 16), 16)], cbuf)
        pltpu.sync_copy(we_h, web)
        k = cbuf[...][0]

        zf = jnp.zeros((16,), jnp.float32)
        zi = jnp.zeros((16,), jnp.int32)

        def idx_clear(i, _):
            sidx[pl.ds(i * 16, 16)] = zi
            return 0

        lax.fori_loop(0, SB // 16, idx_clear, 0)

        def zero_body(r, _):
            rb = pl.multiple_of(r * F, 16)
            for j in range(J):
                acc[pl.ds(rb + 16 * j, 16)] = zf
            return 0

        lax.fori_loop(0, NPT, zero_body, 0)

        sems = (sem0, sem1)

        def start(c, b):
            pltpu.make_async_copy(
                table_h.at[sidx.at[pl.ds(pl.multiple_of(c * CE, 8), CE)]],
                rows.at[b], sems[b]).start()

        wes = [web[pl.ds(16 * j, 16)] for j in range(J)]

        def super_body(si, _):
            sb0 = si * SB
            hb = pl.multiple_of(t * PW + sb0, 16)
            pltpu.sync_copy(psrc.at[pl.ds(hb, SB)], sidx)
            pltpu.sync_copy(pdl.at[pl.ds(hb, SB)], dlb.at[pl.ds(0, SB)])
            pltpu.sync_copy(pattr.at[pl.ds(hb, SB)], atb.at[pl.ds(0, SB)])
            sck = jnp.minimum(SB, k - sb0)
            nch = (sck + (CE - 1)) // CE

            @pl.when(nch > 0)
            def _():
                start(0, 0)

            @pl.when(nch > 1)
            def _():
                start(1, 1)

            def pair_body(c2, _):
                for b in range(2):
                    c = 2 * c2 + b

                    @pl.when(c < nch)
                    def _():
                        pltpu.make_async_copy(
                            table_h.at[sidx.at[pl.ds(pl.multiple_of(c * CE, 8), CE)]],
                            rows.at[b], sems[b]).wait()
                        ce_k = jnp.minimum(CE, sck - c * CE)
                        e0 = c * CE

                        def edge_body(e, _):
                            dl = dlb[pl.ds(e0 + e, 16)][0]
                            a = atb[pl.ds(e0 + e, 16)][0]
                            db = pl.multiple_of(dl * F, 16)
                            gs = [rows[b, e, pl.ds(16 * j, 16)]
                                  for j in range(J)]
                            ms = [jnp.maximum(gs[j] + a * wes[j], 0.0)
                                  for j in range(J)]
                            for j in range(J):
                                plsc.addupdate(acc.at[pl.ds(db + 16 * j, 16)], ms[j])
                            return 0

                        lax.fori_loop(0, ce_k, edge_body, 0)

                        @pl.when(c + 2 < nch)
                        def _():
                            start(c + 2, b)
                return 0

            lax.fori_loop(0, (nch + 1) // 2, pair_body, 0)
            return 0

        lax.fori_loop(0, (k + (SB - 1)) // SB, super_body, 0)
        pltpu.sync_copy(acc, out_h.at[pl.ds(pl.multiple_of(t * NPT * F, 16), NPT * F)])

    return edge_kernel


_edge_128 = _make_edge_kernel(128)
_edge_256 = _make_edge_kernel(256)


def _ln(z, g, bt):
    mu = z.mean(-1, keepdims=True)
    var = z.var(-1, keepdims=True)
    return (z - mu) / jnp.sqrt(var + 1e-5) * g + bt


def _bn(z, g, bt):
    mu = z.mean(0)
    var = z.var(0)
    return (z - mu) / jnp.sqrt(var + 1e-5) * g + bt


def kernel(x, edge_index, edge_attr, batch, params):
    src = edge_index[0]
    dst = edge_index[1]
    attr = edge_attr[:, 0]

    psrc, pdl, pattr, pcnt = _partition(dst, src, attr)

    h = x
    outs = [x]
    for i, p in enumerate(params["convs"]):
        F = h.shape[1]
        ek = _edge_128 if F == 128 else _edge_256
        agg = ek(h + p["be"], psrc, pdl, pattr, pcnt, p["We"][0])
        agg = agg.reshape(NPAD, F)[:N]
        z = (h + agg) @ p["W"] + p["b"]
        z = jax.nn.relu(z)
        h = _ln(z, p["g"], p["bt"])
        outs.append(h)

    xc = jnp.concatenate(outs, axis=-1)
    h1 = _bn(jax.nn.relu(xc @ params["fc1"]["W"] + params["fc1"]["b"]),
             params["fc1"]["g"], params["fc1"]["bt"])
    h2 = _bn(jax.nn.relu(h1 @ params["fc2"]["W"] + params["fc2"]["b"]),
             params["fc2"]["g"], params["fc2"]["bt"])
    edge_probs = jax.nn.sigmoid(h2[src] @ params["pol"]["W"] + params["pol"]["b"])[:, 0]
    sums = jax.ops.segment_sum(h2, batch, num_segments=G)
    counts = jax.ops.segment_sum(jnp.ones((N, 1), jnp.float32), batch, num_segments=G)
    pooled = sums / jnp.maximum(counts, 1.0)
    value = jnp.tanh(pooled @ params["val"]["W"] + params["val"]["b"])[:, 0]
    return edge_probs, value


# speedup vs baseline: 5.4900x; 1.1993x over previous
import functools

import jax
import jax.numpy as jnp
from jax import lax
from jax.experimental import pallas as pl
from jax.experimental.pallas import tpu as pltpu
from jax.experimental.pallas import tpu_sc as plsc

N = 10000
E = 320000
F_IN = 128
C = 256
G = 64

NC = 2
NS = 16
NT = NC * NS
NPT = (N + NT - 1) // NT
NPAD = NT * NPT

CH = 3200
PW = E + CH + 16
CE = 64

_MESH = plsc.VectorSubcoreMesh(core_axis_name="c", subcore_axis_name="s")


def _wid():
    return lax.axis_index("s") * NC + lax.axis_index("c")


@functools.partial(
    pl.kernel,
    out_type=(
        jax.ShapeDtypeStruct((NT * PW,), jnp.int32),
        jax.ShapeDtypeStruct((NT * PW,), jnp.int32),
        jax.ShapeDtypeStruct((NT * PW,), jnp.float32),
        jax.ShapeDtypeStruct((NT * 16,), jnp.int32),
    ),
    mesh=_MESH,
    scratch_types=[
        pltpu.VMEM((CH,), jnp.int32),
        pltpu.VMEM((CH,), jnp.int32),
        pltpu.VMEM((CH,), jnp.float32),
        pltpu.VMEM((CH + 16,), jnp.int32),
        pltpu.VMEM((CH + 16,), jnp.int32),
        pltpu.VMEM((CH + 16,), jnp.float32),
        pltpu.VMEM((16,), jnp.int32),
    ],
    compiler_params=pltpu.CompilerParams(needs_layout_passes=False),
)
def _partition(dst_h, src_h, attr_h, osrc, odl, oattr, ocnt,
               dbuf, sbuf, abuf, st_s, st_d, st_a, cbuf):
    t = _wid()
    lo = t * NPT
    hi = lo + NPT
    zi = jnp.zeros((16,), jnp.int32)
    zf = jnp.zeros((16,), jnp.float32)

    def memset(i, _):
        st_s[pl.ds(i * 16, 16)] = zi
        st_d[pl.ds(i * 16, 16)] = zi
        st_a[pl.ds(i * 16, 16)] = zf
        return 0

    lax.fori_loop(0, (CH + 16) // 16, memset, 0)

    def chunk_body(c, carry):
        off, total = carry
        pltpu.sync_copy(dst_h.at[pl.ds(c * CH, CH)], dbuf)
        pltpu.sync_copy(src_h.at[pl.ds(c * CH, CH)], sbuf)
        pltpu.sync_copy(attr_h.at[pl.ds(c * CH, CH)], abuf)

        def vec_body(v, off):
            d = dbuf[pl.ds(v * 16, 16)]
            s = sbuf[pl.ds(v * 16, 16)]
            a = abuf[pl.ds(v * 16, 16)]
            m = (d >= lo) & (d < hi)
            pc = plsc.cumsum(jnp.where(m, 1, 0).astype(jnp.int32))
            pos = pc + (off - 1)
            plsc.store_scatter(st_s, [pos], s, mask=m)
            plsc.store_scatter(st_d, [pos], d - lo, mask=m)
            plsc.store_scatter(st_a, [pos], a, mask=m)
            return off + pc[15]

        off = lax.fori_loop(0, CH // 16, vec_body, off)
        fl = jnp.bitwise_and(off, -16)
        to = pl.multiple_of(t * PW + total, 16)
        pltpu.sync_copy(st_s.at[pl.ds(0, CH)], osrc.at[pl.ds(to, CH)])
        pltpu.sync_copy(st_d.at[pl.ds(0, CH)], odl.at[pl.ds(to, CH)])
        pltpu.sync_copy(st_a.at[pl.ds(0, CH)], oattr.at[pl.ds(to, CH)])
        rs = st_s[pl.ds(fl, 16)]
        rd = st_d[pl.ds(fl, 16)]
        ra = st_a[pl.ds(fl, 16)]
        st_s[pl.ds(0, 16)] = rs
        st_d[pl.ds(0, 16)] = rd
        st_a[pl.ds(0, 16)] = ra
        return off - fl, total + fl

    off, total = lax.fori_loop(0, E // CH, chunk_body, (jnp.int32(0), jnp.int32(0)))
    to = pl.multiple_of(t * PW + total, 16)
    pltpu.sync_copy(st_s.at[pl.ds(0, 16)], osrc.at[pl.ds(to, 16)])
    pltpu.sync_copy(st_d.at[pl.ds(0, 16)], odl.at[pl.ds(to, 16)])
    pltpu.sync_copy(st_a.at[pl.ds(0, 16)], oattr.at[pl.ds(to, 16)])
    cbuf[...] = jnp.broadcast_to(total + off, (16,))
    pltpu.sync_copy(cbuf, ocnt.at[pl.ds(pl.multiple_of(t * 16, 16), 16)])


SB = 2048
SPC = SB // CE


def _make_edge_kernel(F):
    J = F // 16

    @functools.partial(
        pl.kernel,
        out_type=jax.ShapeDtypeStruct((NPAD * F,), jnp.float32),
        mesh=_MESH,
        scratch_types=[
            pltpu.VMEM((NPT * F,), jnp.float32),
            pltpu.VMEM((SB,), jnp.int32),
            pltpu.VMEM((SB + 16,), jnp.int32),
            pltpu.VMEM((SB + 16,), jnp.float32),
            pltpu.VMEM((2, CE, F), jnp.float32),
            pltpu.VMEM((16,), jnp.int32),
            pltpu.VMEM((F,), jnp.float32),
            pltpu.SemaphoreType.DMA,
            pltpu.SemaphoreType.DMA,
        ],
    )
    def edge_kernel(table_h, psrc, pdl, pattr, cnt_h, we_h, out_h,
                    acc, sidx, dlb, atb, rows, cbuf, web, sem0, sem1):
        t = _wid()
        pltpu.sync_copy(cnt_h.at[pl.ds(pl.multiple_of(t * 16, 16), 16)], cbuf)
        pltpu.sync_copy(we_h, web)
        k = cbuf[...][0]

        zf = jnp.zeros((16,), jnp.float32)
        zi = jnp.zeros((16,), jnp.int32)

        def idx_clear(i, _):
            sidx[pl.ds(i * 16, 16)] = zi
            return 0

        lax.fori_loop(0, SB // 16, idx_clear, 0)

        def zero_body(r, _):
            rb = pl.multiple_of(r * F, 16)
            for j in range(J):
                acc[pl.ds(rb + 16 * j, 16)] = zf
            return 0

        lax.fori_loop(0, NPT, zero_body, 0)

        sems = (sem0, sem1)

        def start(c, b):
            pltpu.make_async_copy(
                table_h.at[sidx.at[pl.ds(pl.multiple_of(c * CE, 8), CE)]],
                rows.at[b], sems[b]).start()

        wes = [web[pl.ds(16 * j, 16)] for j in range(J)]

        def super_body(si, _):
            sb0 = si * SB
            hb = pl.multiple_of(t * PW + sb0, 16)
            pltpu.sync_copy(psrc.at[pl.ds(hb, SB)], sidx)
            pltpu.sync_copy(pdl.at[pl.ds(hb, SB)], dlb.at[pl.ds(0, SB)])
            pltpu.sync_copy(pattr.at[pl.ds(hb, SB)], atb.at[pl.ds(0, SB)])
            sck = jnp.minimum(SB, k - sb0)
            nch = (sck + (CE - 1)) // CE

            @pl.when(nch > 0)
            def _():
                start(0, 0)

            @pl.when(nch > 1)
            def _():
                start(1, 1)

            def pair_body(c2, _):
                for b in range(2):
                    c = 2 * c2 + b

                    @pl.when(c < nch)
                    def _():
                        pltpu.make_async_copy(
                            table_h.at[sidx.at[pl.ds(pl.multiple_of(c * CE, 8), CE)]],
                            rows.at[b], sems[b]).wait()
                        ce_k = jnp.minimum(CE, sck - c * CE)
                        e0 = c * CE

                        def edge_body(e, _):
                            dl = dlb[pl.ds(e0 + e, 16)][0]
                            a = atb[pl.ds(e0 + e, 16)][0]
                            db = pl.multiple_of(dl * F, 16)
                            gs = [rows[b, e, pl.ds(16 * j, 16)]
                                  for j in range(J)]
                            ms = [jnp.maximum(gs[j] + a * wes[j], 0.0)
                                  for j in range(J)]
                            for j in range(J):
                                plsc.addupdate(acc.at[pl.ds(db + 16 * j, 16)], ms[j])
                            return 0

                        lax.fori_loop(0, ce_k, edge_body, 0)

                        @pl.when(c + 2 < nch)
                        def _():
                            start(c + 2, b)
                return 0

            lax.fori_loop(0, (nch + 1) // 2, pair_body, 0)
            return 0

        lax.fori_loop(0, (k + (SB - 1)) // SB, super_body, 0)
        pltpu.sync_copy(acc, out_h.at[pl.ds(pl.multiple_of(t * NPT * F, 16), NPT * F)])

    return edge_kernel


_edge_128 = _make_edge_kernel(128)
_edge_256 = _make_edge_kernel(256)


EPT = E // NT
_SGC = 2048


@functools.partial(
    pl.kernel,
    out_type=jax.ShapeDtypeStruct((E,), jnp.float32),
    mesh=_MESH,
    scratch_types=[
        pltpu.VMEM((N,), jnp.float32),
        pltpu.VMEM((_SGC,), jnp.int32),
        pltpu.VMEM((_SGC,), jnp.float32),
    ],
    compiler_params=pltpu.CompilerParams(needs_layout_passes=False),
)
def _edge_prob(s_h, src_h, out_h, sbuf, idxb, ob):
    t = _wid()
    pltpu.sync_copy(s_h, sbuf)
    done = 0
    for si in range((EPT + _SGC - 1) // _SGC):
        L = min(_SGC, EPT - si * _SGC)
        base = pl.multiple_of(t * EPT + si * _SGC, 16)
        pltpu.sync_copy(src_h.at[pl.ds(base, L)], idxb.at[pl.ds(0, L)])

        def vbody(v, _):
            vb = pl.multiple_of(v * 16, 16)
            iv = idxb[pl.ds(vb, 16)]
            g = plsc.load_gather(sbuf, [iv])
            p = 1.0 / (1.0 + jnp.exp(-g))
            ob[pl.ds(vb, 16)] = p
            return 0

        lax.fori_loop(0, L // 16, vbody, 0)
        pltpu.sync_copy(ob.at[pl.ds(0, L)], out_h.at[pl.ds(base, L)])
        done += L


RB = 400
NGRID = N // RB


def _row_spec(cols):
    return pl.BlockSpec((RB, cols), lambda i: (i, 0))


def _full_spec(r, c):
    return pl.BlockSpec((r, c), lambda i: (0, 0))


def _tbl_body(x_ref, be_ref, t_ref):
    t_ref[...] = x_ref[...] + be_ref[...]


def _make_table0():
    return pl.pallas_call(
        _tbl_body,
        grid=(NGRID,),
        in_specs=[_row_spec(F_IN), _full_spec(1, F_IN)],
        out_specs=_row_spec(F_IN),
        out_shape=jax.ShapeDtypeStruct((N, F_IN), jnp.float32),
    )


def _layer_body(h_ref, agg_ref, w_ref, b_ref, g_ref, bt_ref, ben_ref,
                hout_ref, tout_ref):
    y = h_ref[...] + agg_ref[...]
    z = jnp.dot(y, w_ref[...], preferred_element_type=jnp.float32) + b_ref[...]
    z = jnp.maximum(z, 0.0)
    mu = jnp.mean(z, axis=-1, keepdims=True)
    var = jnp.mean((z - mu) ** 2, axis=-1, keepdims=True)
    hn = (z - mu) / jnp.sqrt(var + 1e-5) * g_ref[...] + bt_ref[...]
    hout_ref[...] = hn
    tout_ref[...] = hn + ben_ref[...]


def _make_layer(F):
    return pl.pallas_call(
        _layer_body,
        grid=(NGRID,),
        in_specs=[_row_spec(F), _row_spec(F), _full_spec(F, C),
                  _full_spec(1, C), _full_spec(1, C), _full_spec(1, C),
                  _full_spec(1, C)],
        out_specs=(_row_spec(C), _row_spec(C)),
        out_shape=(jax.ShapeDtypeStruct((N, C), jnp.float32),
                   jax.ShapeDtypeStruct((N, C), jnp.float32)),
    )


_layer_128 = _make_layer(F_IN)
_layer_256 = _make_layer(C)

XCD = F_IN + 9 * C
D1 = 2 * C


def _fc_body(x_ref, w_ref, b_ref, u_ref, s_ref, q_ref):
    i = pl.program_id(0)
    u = jnp.dot(x_ref[...], w_ref[...], preferred_element_type=jnp.float32)
    u = jnp.maximum(u + b_ref[...], 0.0)
    u_ref[...] = u
    su = jnp.sum(u, axis=0, keepdims=True)
    sq = jnp.sum(u * u, axis=0, keepdims=True)

    @pl.when(i == 0)
    def _():
        s_ref[...] = su
        q_ref[...] = sq

    @pl.when(i > 0)
    def _():
        s_ref[...] += su
        q_ref[...] += sq


def _make_fc(din, dout):
    return pl.pallas_call(
        _fc_body,
        grid=(NGRID,),
        in_specs=[_row_spec(din), _full_spec(din, dout), _full_spec(1, dout)],
        out_specs=(_row_spec(dout), _full_spec(1, dout), _full_spec(1, dout)),
        out_shape=(jax.ShapeDtypeStruct((N, dout), jnp.float32),
                   jax.ShapeDtypeStruct((1, dout), jnp.float32),
                   jax.ShapeDtypeStruct((1, dout), jnp.float32)),
    )


_fc1 = _make_fc(XCD, D1)


def _bn_of(s, q, u):
    mu = s / N
    var = q / N - mu * mu
    return (u - mu) / jnp.sqrt(var + 1e-5)


def _fc2_body(u_ref, s1_ref, q1_ref, g1_ref, bt1_ref, w_ref, b_ref,
              v_ref, s_ref, q_ref):
    i = pl.program_id(0)
    h1 = _bn_of(s1_ref[...], q1_ref[...], u_ref[...]) * g1_ref[...] + bt1_ref[...]
    v = jnp.dot(h1, w_ref[...], preferred_element_type=jnp.float32)
    v = jnp.maximum(v + b_ref[...], 0.0)
    v_ref[...] = v
    su = jnp.sum(v, axis=0, keepdims=True)
    sq = jnp.sum(v * v, axis=0, keepdims=True)

    @pl.when(i == 0)
    def _():
        s_ref[...] = su
        q_ref[...] = sq

    @pl.when(i > 0)
    def _():
        s_ref[...] += su
        q_ref[...] += sq


_fc2 = pl.pallas_call(
    _fc2_body,
    grid=(NGRID,),
    in_specs=[_row_spec(D1), _full_spec(1, D1), _full_spec(1, D1),
              _full_spec(1, D1), _full_spec(1, D1), _full_spec(D1, C),
              _full_spec(1, C)],
    out_specs=(_row_spec(C), _full_spec(1, C), _full_spec(1, C)),
    out_shape=(jax.ShapeDtypeStruct((N, C), jnp.float32),
               jax.ShapeDtypeStruct((1, C), jnp.float32),
               jax.ShapeDtypeStruct((1, C), jnp.float32)),
)


def _head_body(v_ref, s2_ref, q2_ref, g2_ref, bt2_ref, wp_ref, bp_ref,
               batch_ref, s_ref, ps_ref, pc_ref):
    i = pl.program_id(0)
    h2 = _bn_of(s2_ref[...], q2_ref[...], v_ref[...]) * g2_ref[...] + bt2_ref[...]
    s_ref[...] = jnp.dot(h2, wp_ref[...], preferred_element_type=jnp.float32) + bp_ref[...]
    bb = batch_ref[...].reshape(1, RB)
    oh = (bb == lax.broadcasted_iota(jnp.int32, (G, RB), 0)).astype(jnp.float32)
    psum = jnp.dot(oh, h2, preferred_element_type=jnp.float32)
    pcnt = jnp.sum(oh, axis=1, keepdims=True)

    @pl.when(i == 0)
    def _():
        ps_ref[...] = psum
        pc_ref[...] = pcnt

    @pl.when(i > 0)
    def _():
        ps_ref[...] += psum
        pc_ref[...] += pcnt


_head = pl.pallas_call(
    _head_body,
    grid=(NGRID,),
    in_specs=[_row_spec(C), _full_spec(1, C), _full_spec(1, C),
              _full_spec(1, C), _full_spec(1, C), _full_spec(C, 1),
              _full_spec(1, 1), pl.BlockSpec((RB, 1), lambda i: (i, 0))],
    out_specs=(pl.BlockSpec((RB, 1), lambda i: (i, 0)),
               _full_spec(G, C), _full_spec(G, 1)),
    out_shape=(jax.ShapeDtypeStruct((N, 1), jnp.float32),
               jax.ShapeDtypeStruct((G, C), jnp.float32),
               jax.ShapeDtypeStruct((G, 1), jnp.float32)),
)


def _value_body(ps_ref, pc_ref, wv_ref, bv_ref, val_ref):
    pooled = ps_ref[...] / jnp.maximum(pc_ref[...], 1.0)
    val_ref[...] = jnp.tanh(
        jnp.dot(pooled, wv_ref[...], preferred_element_type=jnp.float32)
        + bv_ref[...])


_value = pl.pallas_call(
    _value_body,
    grid=(1,),
    in_specs=[_full_spec(G, C), _full_spec(G, 1), _full_spec(C, 1),
              _full_spec(1, 1)],
    out_specs=pl.BlockSpec((G, 1), lambda i: (0, 0)),
    out_shape=jax.ShapeDtypeStruct((G, 1), jnp.float32),
)


def kernel(x, edge_index, edge_attr, batch, params):
    src = edge_index[0]
    dst = edge_index[1]
    attr = edge_attr[:, 0]

    psrc, pdl, pattr, pcnt = _partition(dst, src, attr)

    convs = params["convs"]
    h = x
    outs = [x]
    table = _make_table0()(x, convs[0]["be"].reshape(1, F_IN))
    for i, p in enumerate(convs):
        F = h.shape[1]
        ek = _edge_128 if F == 128 else _edge_256
        agg = ek(table, psrc, pdl, pattr, pcnt, p["We"][0])
        agg = agg.reshape(NPAD, F)[:N]
        be_next = (convs[i + 1]["be"] if i + 1 < len(convs)
                   else jnp.zeros((C,), jnp.float32))
        lk = _layer_128 if F == 128 else _layer_256
        h, table = lk(h, agg, p["W"], p["b"].reshape(1, C),
                      p["g"].reshape(1, C), p["bt"].reshape(1, C),
                      be_next.reshape(1, C))
        outs.append(h)

    xc = jnp.concatenate(outs, axis=-1)
    f1 = params["fc1"]
    f2 = params["fc2"]
    u, s1, q1 = _fc1(xc, f1["W"], f1["b"].reshape(1, D1))
    v, s2, q2 = _fc2(u, s1, q1, f1["g"].reshape(1, D1), f1["bt"].reshape(1, D1),
                     f2["W"], f2["b"].reshape(1, C))
    s, psum, pcnt2 = _head(v, s2, q2, f2["g"].reshape(1, C),
                           f2["bt"].reshape(1, C), params["pol"]["W"],
                           params["pol"]["b"].reshape(1, 1),
                           batch.reshape(N, 1))
    edge_probs = _edge_prob(s[:, 0], src)
    value = _value(psum, pcnt2, params["val"]["W"],
                   params["val"]["b"].reshape(1, 1))[:, 0]
    return edge_probs, value
